# baseline (device time: 179974 ns/iter reference)
import jax
import jax.numpy as jnp
from jax import lax
from jax.experimental import pallas as pl
from jax.experimental.pallas import tpu as pltpu

N_DEV = 8


def kernel(x, w_mat):
    m, k_shard = x.shape
    _, n = w_mat.shape
    m_per = m // N_DEV

    def body(x_ref, w_ref, out_ref, comm_ref, send_sems, recv_sems):
        my_pos = lax.axis_index("i")
        left = lax.rem(my_pos + N_DEV - 1, N_DEV)
        right = lax.rem(my_pos + 1, N_DEV)

        barrier_sem = pltpu.get_barrier_semaphore()
        for nbr in (left, right):
            pl.semaphore_signal(
                barrier_sem, inc=1,
                device_id=(nbr,), device_id_type=pl.DeviceIdType.MESH,
            )
        pl.semaphore_wait(barrier_sem, 2)

        def local_chunk(c):
            xs = x_ref[pl.ds(c * m_per, m_per), :]
            return jnp.dot(xs, w_ref[:, :], preferred_element_type=jnp.float32)

        comm_ref[0] = local_chunk(lax.rem(my_pos + N_DEV - 1, N_DEV))

        for s in range(N_DEV - 1):
            send_slot = s % 2
            recv_slot = (s + 1) % 2
            rdma = pltpu.make_async_remote_copy(
                src_ref=comm_ref.at[send_slot],
                dst_ref=comm_ref.at[recv_slot],
                send_sem=send_sems.at[s],
                recv_sem=recv_sems.at[s],
                device_id=(right,),
                device_id_type=pl.DeviceIdType.MESH,
            )
            rdma.start()
            c = lax.rem(my_pos + N_DEV - 2 - s, N_DEV)
            contrib = local_chunk(c)
            rdma.wait()
            if s < N_DEV - 2:
                comm_ref[recv_slot] = comm_ref[recv_slot] + contrib
            else:
                out_ref[:, :] = comm_ref[recv_slot] + contrib

    return pl.pallas_call(
        body,
        out_shape=jax.ShapeDtypeStruct((m_per, n), jnp.float32),
        in_specs=[
            pl.BlockSpec(memory_space=pltpu.VMEM),
            pl.BlockSpec(memory_space=pltpu.VMEM),
        ],
        out_specs=pl.BlockSpec(memory_space=pltpu.VMEM),
        scratch_shapes=[
            pltpu.VMEM((2, m_per, n), jnp.float32),
            pltpu.SemaphoreType.DMA((N_DEV - 1,)),
            pltpu.SemaphoreType.DMA((N_DEV - 1,)),
        ],
        compiler_params=pltpu.CompilerParams(collective_id=0),
    )(x, w_mat)


# device time: 104330 ns/iter; 1.7250x vs baseline; 1.7250x over previous
import jax
import jax.numpy as jnp
from jax import lax
from jax.experimental import pallas as pl
from jax.experimental.pallas import tpu as pltpu

N_DEV = 8


def kernel(x, w_mat):
    m, k_shard = x.shape
    _, n = w_mat.shape
    m_per = m // N_DEV
    n_half = n // 2

    def body(x_ref, w_ref, out_ref,
             cw_ref, ccw_ref,
             cw_send_sems, cw_recv_sems, ccw_send_sems, ccw_recv_sems):
        my_pos = lax.axis_index("i")
        left = lax.rem(my_pos + N_DEV - 1, N_DEV)
        right = lax.rem(my_pos + 1, N_DEV)

        barrier_sem = pltpu.get_barrier_semaphore()
        for nbr in (left, right):
            pl.semaphore_signal(
                barrier_sem, inc=1,
                device_id=(nbr,), device_id_type=pl.DeviceIdType.MESH,
            )
        pl.semaphore_wait(barrier_sem, 2)

        def contrib_cw(c):
            xs = x_ref[pl.ds(c * m_per, m_per), :]
            return jnp.dot(xs, w_ref[:, :n_half],
                           preferred_element_type=jnp.float32)

        def contrib_ccw(c):
            xs = x_ref[pl.ds(c * m_per, m_per), :]
            return jnp.dot(xs, w_ref[:, n_half:],
                           preferred_element_type=jnp.float32)

        cw_ref[0] = contrib_cw(lax.rem(my_pos + N_DEV - 1, N_DEV))
        ccw_ref[0] = contrib_ccw(lax.rem(my_pos + 1, N_DEV))

        for s in range(N_DEV - 1):
            send_slot = s % 2
            recv_slot = (s + 1) % 2
            rdma_cw = pltpu.make_async_remote_copy(
                src_ref=cw_ref.at[send_slot],
                dst_ref=cw_ref.at[recv_slot],
                send_sem=cw_send_sems.at[s],
                recv_sem=cw_recv_sems.at[s],
                device_id=(right,),
                device_id_type=pl.DeviceIdType.MESH,
            )
            rdma_ccw = pltpu.make_async_remote_copy(
                src_ref=ccw_ref.at[send_slot],
                dst_ref=ccw_ref.at[recv_slot],
                send_sem=ccw_send_sems.at[s],
                recv_sem=ccw_recv_sems.at[s],
                device_id=(left,),
                device_id_type=pl.DeviceIdType.MESH,
            )
            rdma_cw.start()
            rdma_ccw.start()
            add_cw = contrib_cw(lax.rem(my_pos + N_DEV - 2 - s, N_DEV))
            add_ccw = contrib_ccw(lax.rem(my_pos + 2 + s, N_DEV))
            rdma_cw.wait()
            rdma_ccw.wait()
            if s < N_DEV - 2:
                cw_ref[recv_slot] = cw_ref[recv_slot] + add_cw
                ccw_ref[recv_slot] = ccw_ref[recv_slot] + add_ccw
            else:
                out_ref[:, :n_half] = cw_ref[recv_slot] + add_cw
                out_ref[:, n_half:] = ccw_ref[recv_slot] + add_ccw

    return pl.pallas_call(
        body,
        out_shape=jax.ShapeDtypeStruct((m_per, n), jnp.float32),
        in_specs=[
            pl.BlockSpec(memory_space=pltpu.VMEM),
            pl.BlockSpec(memory_space=pltpu.VMEM),
        ],
        out_specs=pl.BlockSpec(memory_space=pltpu.VMEM),
        scratch_shapes=[
            pltpu.VMEM((2, m_per, n_half), jnp.float32),
            pltpu.VMEM((2, m_per, n_half), jnp.float32),
            pltpu.SemaphoreType.DMA((N_DEV - 1,)),
            pltpu.SemaphoreType.DMA((N_DEV - 1,)),
            pltpu.SemaphoreType.DMA((N_DEV - 1,)),
            pltpu.SemaphoreType.DMA((N_DEV - 1,)),
        ],
        compiler_params=pltpu.CompilerParams(collective_id=0),
    )(x, w_mat)


# device time: 90107 ns/iter; 1.9973x vs baseline; 1.1578x over previous
import jax
import jax.numpy as jnp
from jax import lax
from jax.experimental import pallas as pl
from jax.experimental.pallas import tpu as pltpu

N_DEV = 8
N_STRIPE = 4
N_SLOT = 3
STRIPE_ORDER = (0, 2, 1, 3)


def kernel(x, w_mat):
    m, k_shard = x.shape
    _, n = w_mat.shape
    m_per = m // N_DEV
    nq = n // N_STRIPE

    def body(x_ref, w_ref, out_ref, buf_ref, send_sems, recv_sems):
        my_pos = lax.axis_index("i")
        left = lax.rem(my_pos + N_DEV - 1, N_DEV)
        right = lax.rem(my_pos + 1, N_DEV)

        barrier_sem = pltpu.get_barrier_semaphore()
        for nbr in (left, right):
            pl.semaphore_signal(
                barrier_sem, inc=1,
                device_id=(nbr,), device_id_type=pl.DeviceIdType.MESH,
            )
        pl.semaphore_wait(barrier_sem, 2)

        def contrib(c, q):
            xs = x_ref[pl.ds(c * m_per, m_per), :]
            return jnp.dot(xs, w_ref[:, q * nq:(q + 1) * nq],
                           preferred_element_type=jnp.float32)

        def send_chunk0(q):
            if q < 2:
                return lax.rem(my_pos + N_DEV - 1, N_DEV)
            return lax.rem(my_pos + 1, N_DEV)

        def recv_chunk(q, s):
            if q < 2:
                return lax.rem(my_pos + N_DEV - 2 - s, N_DEV)
            return lax.rem(my_pos + 2 + s, N_DEV)

        prev = [None] * N_STRIPE
        pend = [None] * N_STRIPE

        for s in range(N_DEV - 1):
            send_slot = s % N_SLOT
            recv_slot = (s + 1) % N_SLOT
            for q in STRIPE_ORDER:
                if s == 0:
                    buf_ref[q, 0] = contrib(send_chunk0(q), q)
                else:
                    prev[q].wait()
                    buf_ref[q, send_slot] = buf_ref[q, send_slot] + pend[q]
                rdma = pltpu.make_async_remote_copy(
                    src_ref=buf_ref.at[q, send_slot],
                    dst_ref=buf_ref.at[q, recv_slot],
                    send_sem=send_sems.at[q, s],
                    recv_sem=recv_sems.at[q, s],
                    device_id=(right,) if q < 2 else (left,),
                    device_id_type=pl.DeviceIdType.MESH,
                )
                rdma.start()
                prev[q] = rdma
            for q in STRIPE_ORDER:
                pend[q] = contrib(recv_chunk(q, s), q)

        final_slot = (N_DEV - 1) % N_SLOT
        for q in STRIPE_ORDER:
            prev[q].wait()
            out_ref[:, q * nq:(q + 1) * nq] = buf_ref[q, final_slot] + pend[q]

    return pl.pallas_call(
        body,
        out_shape=jax.ShapeDtypeStruct((m_per, n), jnp.float32),
        in_specs=[
            pl.BlockSpec(memory_space=pltpu.VMEM),
            pl.BlockSpec(memory_space=pltpu.VMEM),
        ],
        out_specs=pl.BlockSpec(memory_space=pltpu.VMEM),
        scratch_shapes=[
            pltpu.VMEM((N_STRIPE, N_SLOT, m_per, nq), jnp.float32),
            pltpu.SemaphoreType.DMA((N_STRIPE, N_DEV - 1)),
            pltpu.SemaphoreType.DMA((N_STRIPE, N_DEV - 1)),
        ],
        compiler_params=pltpu.CompilerParams(collective_id=0),
    )(x, w_mat)


# device time: 89418 ns/iter; 2.0127x vs baseline; 1.0077x over previous
import jax
import jax.numpy as jnp
from jax import lax
from jax.experimental import pallas as pl
from jax.experimental.pallas import tpu as pltpu

N_DEV = 8
N_STRIPE = 4
N_SLOT = 3
STRIPE_ORDER = (0, 2, 1, 3)


def kernel(x, w_mat):
    m, k_shard = x.shape
    _, n = w_mat.shape
    m_per = m // N_DEV
    nq = n // N_STRIPE

    def gray_fwd(k):
        g = k ^ (k >> 1)
        z = g & 1
        y = (g >> 1) & 1
        xb = (g >> 2) & 1
        return 4 * z + 2 * y + (xb ^ y)

    def ring_pos(p):
        z = p >> 2
        r = p & 3
        y = r >> 1
        xb = (r & 1) ^ y
        g = 4 * xb + 2 * y + z
        return g ^ (g >> 1) ^ (g >> 2)

    def body(x_ref, w_ref, out_ref, buf_ref, send_sems, recv_sems):
        my_pos = lax.axis_index("i")
        v = ring_pos(my_pos)
        left = gray_fwd(lax.rem(v + N_DEV - 1, N_DEV))
        right = gray_fwd(lax.rem(v + 1, N_DEV))

        barrier_sem = pltpu.get_barrier_semaphore()
        for nbr in (left, right):
            pl.semaphore_signal(
                barrier_sem, inc=1,
                device_id=(nbr,), device_id_type=pl.DeviceIdType.MESH,
            )
        pl.semaphore_wait(barrier_sem, 2)

        def contrib(c, q):
            xs = x_ref[pl.ds(c * m_per, m_per), :]
            return jnp.dot(xs, w_ref[:, q * nq:(q + 1) * nq],
                           preferred_element_type=jnp.float32)

        def send_chunk0(q):
            if q < 2:
                return gray_fwd(lax.rem(v + N_DEV - 1, N_DEV))
            return gray_fwd(lax.rem(v + 1, N_DEV))

        def recv_chunk(q, s):
            if q < 2:
                return gray_fwd(lax.rem(v + N_DEV - 2 - s, N_DEV))
            return gray_fwd(lax.rem(v + 2 + s, N_DEV))

        prev = [None] * N_STRIPE
        pend = [None] * N_STRIPE

        for s in range(N_DEV - 1):
            send_slot = s % N_SLOT
            recv_slot = (s + 1) % N_SLOT
            for q in STRIPE_ORDER:
                if s == 0:
                    buf_ref[q, 0] = contrib(send_chunk0(q), q)
                else:
                    prev[q].wait()
                    buf_ref[q, send_slot] = buf_ref[q, send_slot] + pend[q]
                rdma = pltpu.make_async_remote_copy(
                    src_ref=buf_ref.at[q, send_slot],
                    dst_ref=buf_ref.at[q, recv_slot],
                    send_sem=send_sems.at[q, s],
                    recv_sem=recv_sems.at[q, s],
                    device_id=(right,) if q < 2 else (left,),
                    device_id_type=pl.DeviceIdType.MESH,
                )
                rdma.start()
                prev[q] = rdma
            for q in STRIPE_ORDER:
                pend[q] = contrib(recv_chunk(q, s), q)

        final_slot = (N_DEV - 1) % N_SLOT
        for q in STRIPE_ORDER:
            prev[q].wait()
            out_ref[:, q * nq:(q + 1) * nq] = buf_ref[q, final_slot] + pend[q]

    return pl.pallas_call(
        body,
        out_shape=jax.ShapeDtypeStruct((m_per, n), jnp.float32),
        in_specs=[
            pl.BlockSpec(memory_space=pltpu.VMEM),
            pl.BlockSpec(memory_space=pltpu.VMEM),
        ],
        out_specs=pl.BlockSpec(memory_space=pltpu.VMEM),
        scratch_shapes=[
            pltpu.VMEM((N_STRIPE, N_SLOT, m_per, nq), jnp.float32),
            pltpu.SemaphoreType.DMA((N_STRIPE, N_DEV - 1)),
            pltpu.SemaphoreType.DMA((N_STRIPE, N_DEV - 1)),
        ],
        compiler_params=pltpu.CompilerParams(collective_id=0),
    )(x, w_mat)


# device time: 73675 ns/iter; 2.4428x vs baseline; 1.2137x over previous
import jax
import jax.numpy as jnp
from jax import lax
from jax.experimental import pallas as pl
from jax.experimental.pallas import tpu as pltpu

N_DEV = 8

G = (0, 1, 3, 2, 4, 5, 7, 6)

AXIS_SLOT_BIT = {"x": 1, "y": 2, "z": 4}
AXIS_DEV_MASK = {"x": 1, "y": 3, "z": 4}

FLOW_ORDERS = (("x", "y", "z"), ("y", "z", "x"), ("z", "x", "y"))
FLOW_COLS = ((0, 768), (768, 1408), (1408, 2048))
LAND_BASE = (0, 4, 6)


def _round_slots(order):
    active = list(range(8))
    rounds = []
    for axis in order:
        bit = AXIS_SLOT_BIT[axis]
        sent = [j for j in active if j & bit]
        kept = [j for j in active if not (j & bit)]
        rounds.append((axis, sent, kept))
        active = kept
    return rounds


def kernel(x, w_mat):
    m, k_shard = x.shape
    _, n = w_mat.shape
    m_per = m // N_DEV

    flow_rounds = [_round_slots(o) for o in FLOW_ORDERS]

    def body(x_ref, w_ref, out_ref,
             acc0, acc1, acc2, land0, land1, land2,
             send_sems, recv_sems):
        accs = (acc0, acc1, acc2)
        lands = (land0, land1, land2)
        my_pos = lax.axis_index("i")

        barrier_sem = pltpu.get_barrier_semaphore()
        for mask in (1, 3, 4):
            pl.semaphore_signal(
                barrier_sem, inc=1,
                device_id=(my_pos ^ mask,),
                device_id_type=pl.DeviceIdType.MESH,
            )
        pl.semaphore_wait(barrier_sem, 3)

        def seed(f, j):
            c = my_pos ^ G[j]
            lo, hi = FLOW_COLS[f]
            xs = x_ref[pl.ds(c * m_per, m_per), :]
            accs[f][j] = jnp.dot(xs, w_ref[:, lo:hi],
                                 preferred_element_type=jnp.float32)

        def start_round(f, r):
            axis, sent, _ = flow_rounds[f][r]
            partner = my_pos ^ AXIS_DEV_MASK[axis]
            rdmas = []
            for i, j in enumerate(sent):
                rdma = pltpu.make_async_remote_copy(
                    src_ref=accs[f].at[j],
                    dst_ref=lands[f].at[LAND_BASE[r] + i],
                    send_sem=send_sems.at[f, r, i],
                    recv_sem=recv_sems.at[f, r, i],
                    device_id=(partner,),
                    device_id_type=pl.DeviceIdType.MESH,
                )
                rdma.start()
                rdmas.append(rdma)
            return rdmas

        def finish_round(f, r, rdmas):
            _, _, kept = flow_rounds[f][r]
            for i, rdma in enumerate(rdmas):
                rdma.wait()
                accs[f][kept[i]] = accs[f][kept[i]] + lands[f][LAND_BASE[r] + i]

        inflight = [None, None, None]
        for f in range(3):
            for j in flow_rounds[f][0][1]:
                seed(f, j)
            inflight[f] = start_round(f, 0)
        for f in range(3):
            for j in flow_rounds[f][0][2]:
                seed(f, j)

        for r in (1, 2):
            for f in (1, 2, 0):
                finish_round(f, r - 1, inflight[f])
                inflight[f] = start_round(f, r)
        for f in (1, 2, 0):
            finish_round(f, 2, inflight[f])
            lo, hi = FLOW_COLS[f]
            out_ref[:, lo:hi] = accs[f][0]

    w0 = FLOW_COLS[0][1] - FLOW_COLS[0][0]
    w1 = FLOW_COLS[1][1] - FLOW_COLS[1][0]
    w2 = FLOW_COLS[2][1] - FLOW_COLS[2][0]
    return pl.pallas_call(
        body,
        out_shape=jax.ShapeDtypeStruct((m_per, n), jnp.float32),
        in_specs=[
            pl.BlockSpec(memory_space=pltpu.VMEM),
            pl.BlockSpec(memory_space=pltpu.VMEM),
        ],
        out_specs=pl.BlockSpec(memory_space=pltpu.VMEM),
        scratch_shapes=[
            pltpu.VMEM((N_DEV, m_per, w0), jnp.float32),
            pltpu.VMEM((N_DEV, m_per, w1), jnp.float32),
            pltpu.VMEM((N_DEV, m_per, w2), jnp.float32),
            pltpu.VMEM((7, m_per, w0), jnp.float32),
            pltpu.VMEM((7, m_per, w1), jnp.float32),
            pltpu.VMEM((7, m_per, w2), jnp.float32),
            pltpu.SemaphoreType.DMA((3, 3, 4)),
            pltpu.SemaphoreType.DMA((3, 3, 4)),
        ],
        compiler_params=pltpu.CompilerParams(collective_id=0),
    )(x, w_mat)


# device time: 73631 ns/iter; 2.4443x vs baseline; 1.0006x over previous
import jax
import jax.numpy as jnp
from jax import lax
from jax.experimental import pallas as pl
from jax.experimental.pallas import tpu as pltpu

N_DEV = 8

G = (0, 1, 3, 2, 4, 5, 7, 6)

AXIS_SLOT_BIT = {"x": 1, "y": 2, "z": 4}
AXIS_DEV_MASK = {"x": 1, "y": 3, "z": 4}

FLOW_ORDERS = (("x", "y", "z"), ("y", "z", "x"), ("z", "x", "y"))
FLOW_COLS = ((0, 768), (768, 1408), (1408, 2048))
LAND_BASE = (0, 4, 6)


def _round_slots(order):
    active = list(range(8))
    rounds = []
    for axis in order:
        bit = AXIS_SLOT_BIT[axis]
        sent = [j for j in active if j & bit]
        kept = [j for j in active if not (j & bit)]
        rounds.append((axis, sent, kept))
        active = kept
    return rounds


def kernel(x, w_mat):
    m, k_shard = x.shape
    _, n = w_mat.shape
    m_per = m // N_DEV

    flow_rounds = [_round_slots(o) for o in FLOW_ORDERS]

    def body(x_ref, w_ref, out_ref,
             acc0, acc1, acc2, land0, land1, land2,
             send_sems, recv_sems):
        accs = (acc0, acc1, acc2)
        lands = (land0, land1, land2)
        my_pos = lax.axis_index("i")

        barrier_sem = pltpu.get_barrier_semaphore()
        for mask in (1, 3, 4):
            pl.semaphore_signal(
                barrier_sem, inc=1,
                device_id=(my_pos ^ mask,),
                device_id_type=pl.DeviceIdType.MESH,
            )
        pl.semaphore_wait(barrier_sem, 3)

        def seed(f, j):
            c = my_pos ^ G[j]
            lo, hi = FLOW_COLS[f]
            xs = x_ref[pl.ds(c * m_per, m_per), :]
            accs[f][j] = jnp.dot(xs, w_ref[:, lo:hi],
                                 preferred_element_type=jnp.float32)

        def start_round(f, r):
            axis, sent, _ = flow_rounds[f][r]
            partner = my_pos ^ AXIS_DEV_MASK[axis]
            rdmas = []
            for i, j in enumerate(sent):
                rdma = pltpu.make_async_remote_copy(
                    src_ref=accs[f].at[j],
                    dst_ref=lands[f].at[LAND_BASE[r] + i],
                    send_sem=send_sems.at[f, r, i],
                    recv_sem=recv_sems.at[f, r, i],
                    device_id=(partner,),
                    device_id_type=pl.DeviceIdType.MESH,
                )
                rdma.start()
                rdmas.append(rdma)
            return rdmas

        def finish_round(f, r, rdmas):
            _, _, kept = flow_rounds[f][r]
            for i, rdma in enumerate(rdmas):
                rdma.wait()
                folded = accs[f][kept[i]] + lands[f][LAND_BASE[r] + i]
                if r == 2:
                    lo, hi = FLOW_COLS[f]
                    out_ref[:, lo:hi] = folded
                else:
                    accs[f][kept[i]] = folded

        inflight = [None, None, None]
        for f in range(3):
            for j in flow_rounds[f][0][1]:
                seed(f, j)
            inflight[f] = start_round(f, 0)
        for f in range(3):
            for j in flow_rounds[f][0][2]:
                seed(f, j)

        for r in (1, 2):
            for f in (1, 2, 0):
                finish_round(f, r - 1, inflight[f])
                inflight[f] = start_round(f, r)
        for f in (1, 2, 0):
            finish_round(f, 2, inflight[f])

    w0 = FLOW_COLS[0][1] - FLOW_COLS[0][0]
    w1 = FLOW_COLS[1][1] - FLOW_COLS[1][0]
    w2 = FLOW_COLS[2][1] - FLOW_COLS[2][0]
    return pl.pallas_call(
        body,
        out_shape=jax.ShapeDtypeStruct((m_per, n), jnp.float32),
        in_specs=[
            pl.BlockSpec(memory_space=pltpu.VMEM),
            pl.BlockSpec(memory_space=pltpu.VMEM),
        ],
        out_specs=pl.BlockSpec(memory_space=pltpu.VMEM),
        scratch_shapes=[
            pltpu.VMEM((N_DEV, m_per, w0), jnp.float32),
            pltpu.VMEM((N_DEV, m_per, w1), jnp.float32),
            pltpu.VMEM((N_DEV, m_per, w2), jnp.float32),
            pltpu.VMEM((7, m_per, w0), jnp.float32),
            pltpu.VMEM((7, m_per, w1), jnp.float32),
            pltpu.VMEM((7, m_per, w2), jnp.float32),
            pltpu.SemaphoreType.DMA((3, 3, 4)),
            pltpu.SemaphoreType.DMA((3, 3, 4)),
        ],
        compiler_params=pltpu.CompilerParams(collective_id=0),
    )(x, w_mat)


# device time: 70964 ns/iter; 2.5361x vs baseline; 1.0376x over previous
import jax
import jax.numpy as jnp
from jax import lax
from jax.experimental import pallas as pl
from jax.experimental.pallas import tpu as pltpu

N_DEV = 8

G = (0, 1, 3, 2, 4, 5, 7, 6)

AXIS_SLOT_BIT = {"x": 1, "y": 2, "z": 4}
AXIS_DEV_MASK = {"x": 1, "y": 3, "z": 4}

FLOW_ORDERS = (
    ("x", "y", "z"),
    ("y", "z", "x"),
    ("z", "x", "y"),
    ("y", "x", "z"),
)
FLOW_COLS = ((0, 640), (640, 1280), (1280, 1920), (1920, 2048))
N_FLOWS = len(FLOW_ORDERS)
FLOW_SCHED = (0, 2, 1, 3)
LAND_BASE = (0, 4, 6)


def _round_slots(order):
    active = list(range(8))
    rounds = []
    for axis in order:
        bit = AXIS_SLOT_BIT[axis]
        sent = [j for j in active if j & bit]
        kept = [j for j in active if not (j & bit)]
        rounds.append((axis, sent, kept))
        active = kept
    return rounds


def kernel(x, w_mat):
    m, k_shard = x.shape
    _, n = w_mat.shape
    m_per = m // N_DEV

    flow_rounds = [_round_slots(o) for o in FLOW_ORDERS]

    def body(x_ref, w_ref, out_ref,
             acc0, acc1, acc2, acc3, land0, land1, land2, land3,
             send_sems, recv_sems):
        accs = (acc0, acc1, acc2, acc3)
        lands = (land0, land1, land2, land3)
        my_pos = lax.axis_index("i")

        barrier_sem = pltpu.get_barrier_semaphore()
        for mask in (1, 3, 4):
            pl.semaphore_signal(
                barrier_sem, inc=1,
                device_id=(my_pos ^ mask,),
                device_id_type=pl.DeviceIdType.MESH,
            )
        pl.semaphore_wait(barrier_sem, 3)

        def seed(f, j):
            c = my_pos ^ G[j]
            lo, hi = FLOW_COLS[f]
            xs = x_ref[pl.ds(c * m_per, m_per), :]
            accs[f][j] = jnp.dot(xs, w_ref[:, lo:hi],
                                 preferred_element_type=jnp.float32)

        def start_round(f, r):
            axis, sent, _ = flow_rounds[f][r]
            partner = my_pos ^ AXIS_DEV_MASK[axis]
            rdmas = []
            for i, j in enumerate(sent):
                rdma = pltpu.make_async_remote_copy(
                    src_ref=accs[f].at[j],
                    dst_ref=lands[f].at[LAND_BASE[r] + i],
                    send_sem=send_sems.at[f, r, i],
                    recv_sem=recv_sems.at[f, r, i],
                    device_id=(partner,),
                    device_id_type=pl.DeviceIdType.MESH,
                )
                rdma.start()
                rdmas.append(rdma)
            return rdmas

        def finish_round(f, r, rdmas):
            _, _, kept = flow_rounds[f][r]
            for i, rdma in enumerate(rdmas):
                rdma.wait()
                folded = accs[f][kept[i]] + lands[f][LAND_BASE[r] + i]
                if r == 2:
                    lo, hi = FLOW_COLS[f]
                    out_ref[:, lo:hi] = folded
                else:
                    accs[f][kept[i]] = folded

        inflight = [None] * N_FLOWS
        for f in FLOW_SCHED:
            for j in flow_rounds[f][0][1]:
                seed(f, j)
            inflight[f] = start_round(f, 0)
        for f in FLOW_SCHED:
            for j in flow_rounds[f][0][2]:
                seed(f, j)

        for r in (1, 2):
            for f in FLOW_SCHED:
                finish_round(f, r - 1, inflight[f])
                inflight[f] = start_round(f, r)
        for f in FLOW_SCHED:
            finish_round(f, 2, inflight[f])

    widths = [hi - lo for lo, hi in FLOW_COLS]
    return pl.pallas_call(
        body,
        out_shape=jax.ShapeDtypeStruct((m_per, n), jnp.float32),
        in_specs=[
            pl.BlockSpec(memory_space=pltpu.VMEM),
            pl.BlockSpec(memory_space=pltpu.VMEM),
        ],
        out_specs=pl.BlockSpec(memory_space=pltpu.VMEM),
        scratch_shapes=(
            [pltpu.VMEM((N_DEV, m_per, w), jnp.float32) for w in widths]
            + [pltpu.VMEM((7, m_per, w), jnp.float32) for w in widths]
            + [
                pltpu.SemaphoreType.DMA((N_FLOWS, 3, 4)),
                pltpu.SemaphoreType.DMA((N_FLOWS, 3, 4)),
            ]
        ),
        compiler_params=pltpu.CompilerParams(collective_id=0),
    )(x, w_mat)


# device time: 66262 ns/iter; 2.7161x vs baseline; 1.0710x over previous
import jax
import jax.numpy as jnp
from jax import lax
from jax.experimental import pallas as pl
from jax.experimental.pallas import tpu as pltpu

N_DEV = 8

G = (0, 1, 3, 2, 4, 5, 7, 6)

AXIS_SLOT_BIT = {"x": 1, "y": 2, "z": 4}
AXIS_DEV_MASK = {"x": 1, "y": 3, "z": 4}

FLOW_ORDERS = (
    ("x", "y", "z"),
    ("y", "z", "x"),
    ("z", "x", "y"),
    ("y", "x", "z"),
)
FLOW_COLS = ((0, 640), (640, 1280), (1280, 1920), (1920, 2048))
N_FLOWS = len(FLOW_ORDERS)
FLOW_SCHED = (0, 2, 1, 3)
LAND_BASE = (0, 4, 6)


def _round_slots(order):
    active = list(range(8))
    rounds = []
    bits = [AXIS_SLOT_BIT[a] for a in order]
    for r, axis in enumerate(order):
        bit = bits[r]
        sent = [j for j in active if j & bit]
        if r + 1 < len(bits):
            nbit = bits[r + 1]
            sent.sort(key=lambda j: 0 if (j ^ bit) & nbit else 1)
        kept = [j ^ bit for j in sent]
        rounds.append((axis, sent, kept))
        active = sorted(kept)
    return rounds


def kernel(x, w_mat):
    m, k_shard = x.shape
    _, n = w_mat.shape
    m_per = m // N_DEV

    flow_rounds = [_round_slots(o) for o in FLOW_ORDERS]

    def body(x_ref, w_ref, out_ref,
             acc0, acc1, acc2, acc3, land0, land1, land2, land3,
             send_sems, recv_sems):
        accs = (acc0, acc1, acc2, acc3)
        lands = (land0, land1, land2, land3)
        my_pos = lax.axis_index("i")

        barrier_sem = pltpu.get_barrier_semaphore()
        for mask in (1, 3, 4):
            pl.semaphore_signal(
                barrier_sem, inc=1,
                device_id=(my_pos ^ mask,),
                device_id_type=pl.DeviceIdType.MESH,
            )
        pl.semaphore_wait(barrier_sem, 3)

        def seed(f, j):
            c = my_pos ^ G[j]
            lo, hi = FLOW_COLS[f]
            xs = x_ref[pl.ds(c * m_per, m_per), :]
            accs[f][j] = jnp.dot(xs, w_ref[:, lo:hi],
                                 preferred_element_type=jnp.float32)

        def start_round(f, r):
            axis, sent, _ = flow_rounds[f][r]
            partner = my_pos ^ AXIS_DEV_MASK[axis]
            rdmas = []
            for i, j in enumerate(sent):
                rdma = pltpu.make_async_remote_copy(
                    src_ref=accs[f].at[j],
                    dst_ref=lands[f].at[LAND_BASE[r] + i],
                    send_sem=send_sems.at[f, r, i],
                    recv_sem=recv_sems.at[f, r, i],
                    device_id=(partner,),
                    device_id_type=pl.DeviceIdType.MESH,
                )
                rdma.start()
                rdmas.append(rdma)
            return rdmas

        def fold(f, r, rdmas, i):
            _, _, kept = flow_rounds[f][r]
            rdmas[i].wait()
            folded = accs[f][kept[i]] + lands[f][LAND_BASE[r] + i]
            if r == 2:
                lo, hi = FLOW_COLS[f]
                out_ref[:, lo:hi] = folded
            else:
                accs[f][kept[i]] = folded

        inflight = [None] * N_FLOWS
        for f in FLOW_SCHED:
            for j in flow_rounds[f][0][1]:
                seed(f, j)
            inflight[f] = start_round(f, 0)
        for f in FLOW_SCHED:
            for j in flow_rounds[f][0][2]:
                seed(f, j)

        for r in (1, 2):
            prev = list(inflight)
            for f in FLOW_SCHED:
                n_need = len(flow_rounds[f][r][1])
                for i in range(n_need):
                    fold(f, r - 1, prev[f], i)
                inflight[f] = start_round(f, r)
            for f in FLOW_SCHED:
                n_need = len(flow_rounds[f][r][1])
                for i in range(n_need, len(prev[f])):
                    fold(f, r - 1, prev[f], i)
        for f in FLOW_SCHED:
            fold(f, 2, inflight[f], 0)

    widths = [hi - lo for lo, hi in FLOW_COLS]
    return pl.pallas_call(
        body,
        out_shape=jax.ShapeDtypeStruct((m_per, n), jnp.float32),
        in_specs=[
            pl.BlockSpec(memory_space=pltpu.VMEM),
            pl.BlockSpec(memory_space=pltpu.VMEM),
        ],
        out_specs=pl.BlockSpec(memory_space=pltpu.VMEM),
        scratch_shapes=(
            [pltpu.VMEM((N_DEV, m_per, w), jnp.float32) for w in widths]
            + [pltpu.VMEM((7, m_per, w), jnp.float32) for w in widths]
            + [
                pltpu.SemaphoreType.DMA((N_FLOWS, 3, 4)),
                pltpu.SemaphoreType.DMA((N_FLOWS, 3, 4)),
            ]
        ),
        compiler_params=pltpu.CompilerParams(collective_id=0),
    )(x, w_mat)
